# Initial kernel scaffold; baseline (speedup 1.0000x reference)
import jax
import jax.numpy as jnp
from jax.experimental import pallas as pl


def kernel(x, w1, b1, g1, be1, w2, b2, g2, be2, lw1, lg1, lb1, lw2, lg2, lb2):
    raise NotImplementedError("write your pallas kernel here")


# speedup vs baseline: 8.0392x; 8.0392x over previous
import functools

import jax
import jax.numpy as jnp
from jax import lax
from jax.experimental import pallas as pl
from jax.experimental.pallas import tpu as pltpu
from jax.experimental.pallas import tpu_sc as plsc

_EPS = 1e-5


def _y1_body(x_ref, w1_ref, b1_ref, y1_ref, s_ref):
    i = pl.program_id(0)
    y = lax.dot_general(x_ref[...], w1_ref[...], (((1,), (1,)), ((), ())),
                        preferred_element_type=jnp.float32) + b1_ref[...]
    y1_ref[...] = y
    s = jnp.sum(y, axis=0, keepdims=True)

    @pl.when(i == 0)
    def _():
        s_ref[...] = s

    @pl.when(i > 0)
    def _():
        s_ref[...] += s


def _var_body(y_ref, m_ref, sv_ref):
    i = pl.program_id(0)
    c = y_ref[...] - m_ref[...]
    s = jnp.sum(c * c, axis=0, keepdims=True)

    @pl.when(i == 0)
    def _():
        sv_ref[...] = s

    @pl.when(i > 0)
    def _():
        sv_ref[...] += s


def _y2_body(y1_ref, sc_ref, sh_ref, w2_ref, b2_ref, y2_ref, s_ref):
    i = pl.program_id(0)
    h1 = jnp.maximum(y1_ref[...] * sc_ref[...] + sh_ref[...], 0.0)
    y2 = lax.dot_general(h1, w2_ref[...], (((1,), (1,)), ((), ())),
                         preferred_element_type=jnp.float32) + b2_ref[...]
    y2_ref[...] = y2
    s = jnp.sum(y2, axis=0, keepdims=True)

    @pl.when(i == 0)
    def _():
        s_ref[...] = s

    @pl.when(i > 0)
    def _():
        s_ref[...] += s


def _pq_body(y2_ref, sc_ref, sh_ref, a_ref, bm_ref, p_ref, q_ref):
    h = jnp.maximum(y2_ref[...] * sc_ref[...] + sh_ref[...], 0.0)
    p_ref[...] = lax.dot_general(h, a_ref[...], (((1,), (1,)), ((), ())),
                                 preferred_element_type=jnp.float32)
    q_ref[...] = lax.dot_general(h, bm_ref[...], (((1,), (1,)), ((), ())),
                                 preferred_element_type=jnp.float32)


def _topk_body(xq_ref, xk_ref, idx_ref, *, tn, n, kk):
    b = pl.program_id(0)
    xq = xq_ref[0]
    xk = xk_ref[0]
    sqq = (xq[:, 0:1] * xq[:, 0:1] + xq[:, 1:2] * xq[:, 1:2]
           + xq[:, 2:3] * xq[:, 2:3])
    sqk = (xk[:, 0] * xk[:, 0] + xk[:, 1] * xk[:, 1]
           + xk[:, 2] * xk[:, 2])[None, :]
    dots = lax.dot_general(xq, xk, (((1,), (1,)), ((), ())),
                           preferred_element_type=jnp.float32)
    d = sqq + sqk - 2.0 * dots
    lane = lax.broadcasted_iota(jnp.int32, (tn, n), 1)
    kcol = lax.broadcasted_iota(jnp.int32, (tn, kk), 1)
    acc = jnp.zeros((tn, kk), jnp.int32)
    for j in range(kk):
        v = jnp.min(d, axis=1, keepdims=True)
        am = jnp.min(jnp.where(d == v, lane, n), axis=1, keepdims=True)
        acc = jnp.where(kcol == j, am, acc)
        d = jnp.where(lane == am, 1e30, d)
    idx_ref[0] = acc + b * n


def _s3_body(gh_ref, q_ref, s_ref, ss_ref, *, tn, kk, do):
    i = pl.program_id(0)
    y = gh_ref[...]
    y = (y.reshape(tn, kk, do) + q_ref[...][:, None, :]).reshape(tn * kk, do)
    s = jnp.sum(y, axis=0, keepdims=True)
    ss = jnp.sum(y * y, axis=0, keepdims=True)

    @pl.when(i == 0)
    def _():
        s_ref[...] = s
        ss_ref[...] = ss

    @pl.when(i > 0)
    def _():
        s_ref[...] += s
        ss_ref[...] += ss


def _main_body(gh_ref, q_ref, sc3_ref, sh3_ref, w_ref,
               mx_ref, mn_ref, s4_ref, ss4_ref, *, tn, kk, do):
    i = pl.program_id(0)
    y = gh_ref[...]
    y = (y.reshape(tn, kk, do) + q_ref[...][:, None, :]).reshape(tn * kk, do)
    z = jnp.maximum(y * sc3_ref[...] + sh3_ref[...], 0.0)
    y4 = lax.dot_general(z, w_ref[...], (((1,), (1,)), ((), ())),
                         preferred_element_type=jnp.float32)
    s = jnp.sum(y4, axis=0, keepdims=True)
    ss = jnp.sum(y4 * y4, axis=0, keepdims=True)
    y43 = y4.reshape(tn, kk, do)
    m = y43[:, 0, :]
    mn = y43[:, 0, :]
    for j in range(1, kk):
        m = jnp.maximum(m, y43[:, j, :])
        mn = jnp.minimum(mn, y43[:, j, :])
    mx_ref[...] = m
    mn_ref[...] = mn

    @pl.when(i == 0)
    def _():
        s4_ref[...] = s
        ss4_ref[...] = ss

    @pl.when(i > 0)
    def _():
        s4_ref[...] += s
        ss4_ref[...] += ss


def _fin_body(mx_ref, mn_ref, sc_ref, sh_ref, o_ref):
    sc = sc_ref[...]
    val = jnp.where(sc >= 0.0, mx_ref[...], mn_ref[...])
    o_ref[...] = jnp.maximum(val * sc + sh_ref[...], 0.0)


def _sc_gather(idx4, h2d, dh):
    nw, sup, inner, ch = 32, 64, 4, 128
    mesh = plsc.VectorSubcoreMesh(core_axis_name="c", subcore_axis_name="s")

    @functools.partial(
        pl.kernel, mesh=mesh,
        out_type=jax.ShapeDtypeStruct((nw, sup, inner, ch, dh), jnp.float32),
        scratch_types=[
            pltpu.VMEM((sup * inner, ch), jnp.int32),
            pltpu.VMEM((inner, ch, dh), jnp.float32),
            pltpu.SemaphoreType.DMA,
        ],
    )
    def gather_k(idx_hbm, tab_hbm, out_hbm, idx_v, buf_v, sem):
        wid = lax.axis_index("s") * 2 + lax.axis_index("c")
        pltpu.sync_copy(idx_hbm.at[wid], idx_v)

        def body(scn, carry):
            descs = [
                pltpu.async_copy(tab_hbm.at[idx_v.at[scn * inner + j]],
                                 buf_v.at[j], sem)
                for j in range(inner)
            ]
            for dsc in descs:
                dsc.wait()
            pltpu.sync_copy(buf_v, out_hbm.at[wid, scn])
            return carry

        lax.fori_loop(0, sup, body, 0)

    return gather_k(idx4, h2d)


def kernel(x, w1, b1, g1, be1, w2, b2, g2, be2, lw1, lg1, lb1, lw2, lg2, lb2):
    bsz, n, din = x.shape
    dh = w1.shape[0]
    do = lw1.shape[0]
    kk = 32
    m = bsz * n
    nm = m * kk
    r = 1024
    tn = 256
    tk = tn * kk

    x2d = x.reshape(m, din)

    y1a, s1a = pl.pallas_call(
        _y1_body,
        grid=(m // r,),
        in_specs=[pl.BlockSpec((r, din), lambda i: (i, 0)),
                  pl.BlockSpec((dh, din), lambda i: (0, 0)),
                  pl.BlockSpec((1, dh), lambda i: (0, 0))],
        out_specs=[pl.BlockSpec((r, dh), lambda i: (i, 0)),
                   pl.BlockSpec((1, dh), lambda i: (0, 0))],
        out_shape=[jax.ShapeDtypeStruct((m, dh), jnp.float32),
                   jax.ShapeDtypeStruct((1, dh), jnp.float32)],
    )(x2d, w1, b1[None, :])
    m1 = s1a / m

    sv1 = pl.pallas_call(
        _var_body,
        grid=(m // r,),
        in_specs=[pl.BlockSpec((r, dh), lambda i: (i, 0)),
                  pl.BlockSpec((1, dh), lambda i: (0, 0))],
        out_specs=pl.BlockSpec((1, dh), lambda i: (0, 0)),
        out_shape=jax.ShapeDtypeStruct((1, dh), jnp.float32),
    )(y1a, m1)
    v1 = sv1 / m
    sc1 = g1[None, :] / jnp.sqrt(v1 + _EPS)
    sh1 = be1[None, :] - m1 * sc1

    y2a, s2a = pl.pallas_call(
        _y2_body,
        grid=(m // r,),
        in_specs=[pl.BlockSpec((r, dh), lambda i: (i, 0)),
                  pl.BlockSpec((1, dh), lambda i: (0, 0)),
                  pl.BlockSpec((1, dh), lambda i: (0, 0)),
                  pl.BlockSpec((dh, dh), lambda i: (0, 0)),
                  pl.BlockSpec((1, dh), lambda i: (0, 0))],
        out_specs=[pl.BlockSpec((r, dh), lambda i: (i, 0)),
                   pl.BlockSpec((1, dh), lambda i: (0, 0))],
        out_shape=[jax.ShapeDtypeStruct((m, dh), jnp.float32),
                   jax.ShapeDtypeStruct((1, dh), jnp.float32)],
    )(y1a, sc1, sh1, w2, b2[None, :])
    m2 = s2a / m

    sv2 = pl.pallas_call(
        _var_body,
        grid=(m // r,),
        in_specs=[pl.BlockSpec((r, dh), lambda i: (i, 0)),
                  pl.BlockSpec((1, dh), lambda i: (0, 0))],
        out_specs=pl.BlockSpec((1, dh), lambda i: (0, 0)),
        out_shape=jax.ShapeDtypeStruct((1, dh), jnp.float32),
    )(y2a, m2)
    v2 = sv2 / m
    sc2 = g2[None, :] / jnp.sqrt(v2 + _EPS)
    sh2 = be2[None, :] - m2 * sc2

    a_mat = lw1[:, :dh]
    bm_mat = lw1[:, dh:] - lw1[:, :dh]

    p2d, q = pl.pallas_call(
        _pq_body,
        grid=(m // r,),
        in_specs=[pl.BlockSpec((r, dh), lambda i: (i, 0)),
                  pl.BlockSpec((1, dh), lambda i: (0, 0)),
                  pl.BlockSpec((1, dh), lambda i: (0, 0)),
                  pl.BlockSpec((do, dh), lambda i: (0, 0)),
                  pl.BlockSpec((do, dh), lambda i: (0, 0))],
        out_specs=[pl.BlockSpec((r, do), lambda i: (i, 0)),
                   pl.BlockSpec((r, do), lambda i: (i, 0))],
        out_shape=[jax.ShapeDtypeStruct((m, do), jnp.float32),
                   jax.ShapeDtypeStruct((m, do), jnp.float32)],
    )(y2a, sc2, sh2, a_mat, bm_mat)

    idxg = pl.pallas_call(
        functools.partial(_topk_body, tn=tn, n=n, kk=kk),
        grid=(bsz, n // tn),
        in_specs=[pl.BlockSpec((1, tn, din), lambda b, t: (b, t, 0)),
                  pl.BlockSpec((1, n, din), lambda b, t: (b, 0, 0))],
        out_specs=pl.BlockSpec((1, tn, kk), lambda b, t: (b, t, 0)),
        out_shape=jax.ShapeDtypeStruct((bsz, n, kk), jnp.int32),
    )(x, x)

    idx4 = idxg.reshape(32, 256, 128)
    gh = _sc_gather(idx4, p2d, do).reshape(nm, do)

    s3, ss3 = pl.pallas_call(
        functools.partial(_s3_body, tn=tn, kk=kk, do=do),
        grid=(nm // tk,),
        in_specs=[pl.BlockSpec((tk, do), lambda i: (i, 0)),
                  pl.BlockSpec((tn, do), lambda i: (i, 0))],
        out_specs=[pl.BlockSpec((1, do), lambda i: (0, 0)),
                   pl.BlockSpec((1, do), lambda i: (0, 0))],
        out_shape=[jax.ShapeDtypeStruct((1, do), jnp.float32),
                   jax.ShapeDtypeStruct((1, do), jnp.float32)],
    )(gh, q)

    mean3 = s3[0] / nm
    var3 = ss3[0] / nm - mean3 * mean3
    sc3 = (lg1 / jnp.sqrt(var3 + _EPS))[None, :]
    sh3 = (lb1 - mean3 * sc3[0])[None, :]

    mxa, mna, s4, ss4 = pl.pallas_call(
        functools.partial(_main_body, tn=tn, kk=kk, do=do),
        grid=(nm // tk,),
        in_specs=[pl.BlockSpec((tk, do), lambda i: (i, 0)),
                  pl.BlockSpec((tn, do), lambda i: (i, 0)),
                  pl.BlockSpec((1, do), lambda i: (0, 0)),
                  pl.BlockSpec((1, do), lambda i: (0, 0)),
                  pl.BlockSpec((do, do), lambda i: (0, 0))],
        out_specs=[pl.BlockSpec((tn, do), lambda i: (i, 0)),
                   pl.BlockSpec((tn, do), lambda i: (i, 0)),
                   pl.BlockSpec((1, do), lambda i: (0, 0)),
                   pl.BlockSpec((1, do), lambda i: (0, 0))],
        out_shape=[jax.ShapeDtypeStruct((m, do), jnp.float32),
                   jax.ShapeDtypeStruct((m, do), jnp.float32),
                   jax.ShapeDtypeStruct((1, do), jnp.float32),
                   jax.ShapeDtypeStruct((1, do), jnp.float32)],
    )(gh, q, sc3, sh3, lw2)

    mean4 = s4[0] / nm
    var4 = ss4[0] / nm - mean4 * mean4
    sc4 = (lg2 / jnp.sqrt(var4 + _EPS))[None, :]
    sh4 = (lb2 - mean4 * sc4[0])[None, :]

    out = pl.pallas_call(
        _fin_body,
        grid=(m // r,),
        in_specs=[pl.BlockSpec((r, do), lambda i: (i, 0)),
                  pl.BlockSpec((r, do), lambda i: (i, 0)),
                  pl.BlockSpec((1, do), lambda i: (0, 0)),
                  pl.BlockSpec((1, do), lambda i: (0, 0))],
        out_specs=pl.BlockSpec((r, do), lambda i: (i, 0)),
        out_shape=jax.ShapeDtypeStruct((m, do), jnp.float32),
    )(mxa, mna, sc4, sh4)

    return out.reshape(bsz, n, do)


# speedup vs baseline: 8.1299x; 1.0113x over previous
import functools

import jax
import jax.numpy as jnp
from jax import lax
from jax.experimental import pallas as pl
from jax.experimental.pallas import tpu as pltpu
from jax.experimental.pallas import tpu_sc as plsc

_EPS = 1e-5


def _y1_body(x_ref, w1_ref, b1_ref, y1_ref, s_ref):
    i = pl.program_id(0)
    y = lax.dot_general(x_ref[...], w1_ref[...], (((1,), (1,)), ((), ())),
                        preferred_element_type=jnp.float32) + b1_ref[...]
    y1_ref[...] = y
    s = jnp.sum(y, axis=0, keepdims=True)

    @pl.when(i == 0)
    def _():
        s_ref[...] = s

    @pl.when(i > 0)
    def _():
        s_ref[...] += s


def _var_body(y_ref, m_ref, sv_ref):
    i = pl.program_id(0)
    c = y_ref[...] - m_ref[...]
    s = jnp.sum(c * c, axis=0, keepdims=True)

    @pl.when(i == 0)
    def _():
        sv_ref[...] = s

    @pl.when(i > 0)
    def _():
        sv_ref[...] += s


def _y2_body(y1_ref, sc_ref, sh_ref, w2_ref, b2_ref, y2_ref, s_ref):
    i = pl.program_id(0)
    h1 = jnp.maximum(y1_ref[...] * sc_ref[...] + sh_ref[...], 0.0)
    y2 = lax.dot_general(h1, w2_ref[...], (((1,), (1,)), ((), ())),
                         preferred_element_type=jnp.float32) + b2_ref[...]
    y2_ref[...] = y2
    s = jnp.sum(y2, axis=0, keepdims=True)

    @pl.when(i == 0)
    def _():
        s_ref[...] = s

    @pl.when(i > 0)
    def _():
        s_ref[...] += s


def _pq_body(y2_ref, sc_ref, sh_ref, a_ref, bm_ref, p_ref, q_ref):
    h = jnp.maximum(y2_ref[...] * sc_ref[...] + sh_ref[...], 0.0)
    p_ref[...] = lax.dot_general(h, a_ref[...], (((1,), (1,)), ((), ())),
                                 preferred_element_type=jnp.float32)
    q_ref[...] = lax.dot_general(h, bm_ref[...], (((1,), (1,)), ((), ())),
                                 preferred_element_type=jnp.float32)


def _topk_body(xq_ref, xk_ref, idx_ref, *, tn, n, kk):
    b = pl.program_id(0)
    xq = xq_ref[0]
    xk = xk_ref[0]
    sqq = (xq[:, 0:1] * xq[:, 0:1] + xq[:, 1:2] * xq[:, 1:2]
           + xq[:, 2:3] * xq[:, 2:3])
    sqk = (xk[:, 0] * xk[:, 0] + xk[:, 1] * xk[:, 1]
           + xk[:, 2] * xk[:, 2])[None, :]
    dots = lax.dot_general(xq, xk, (((1,), (1,)), ((), ())),
                           preferred_element_type=jnp.float32)
    d = sqq + sqk - 2.0 * dots
    lane = lax.broadcasted_iota(jnp.int32, (tn, n), 1)
    kcol = lax.broadcasted_iota(jnp.int32, (tn, kk), 1)
    acc = jnp.zeros((tn, kk), jnp.int32)
    for j in range(kk):
        v = jnp.min(d, axis=1, keepdims=True)
        e = jnp.where(d == v, lane, n)
        am = jnp.min(e, axis=1, keepdims=True)
        acc = jnp.where(kcol == j, am, acc)
        if j + 1 < kk:
            d = jnp.where(e == am, 1e30, d)
    idx_ref[0] = acc + b * n


def _s3_body(gh_ref, q_ref, s_ref, ss_ref, *, tn, kk, do):
    i = pl.program_id(0)
    y = gh_ref[...]
    y = (y.reshape(tn, kk, do) + q_ref[...][:, None, :]).reshape(tn * kk, do)
    s = jnp.sum(y, axis=0, keepdims=True)
    ss = jnp.sum(y * y, axis=0, keepdims=True)

    @pl.when(i == 0)
    def _():
        s_ref[...] = s
        ss_ref[...] = ss

    @pl.when(i > 0)
    def _():
        s_ref[...] += s
        ss_ref[...] += ss


def _main_body(gh_ref, q_ref, sc3_ref, sh3_ref, w_ref,
               mx_ref, mn_ref, s4_ref, ss4_ref, *, tn, kk, do):
    i = pl.program_id(0)
    y = gh_ref[...]
    y = (y.reshape(tn, kk, do) + q_ref[...][:, None, :]).reshape(tn * kk, do)
    z = jnp.maximum(y * sc3_ref[...] + sh3_ref[...], 0.0)
    y4 = lax.dot_general(z, w_ref[...], (((1,), (1,)), ((), ())),
                         preferred_element_type=jnp.float32)
    s = jnp.sum(y4, axis=0, keepdims=True)
    ss = jnp.sum(y4 * y4, axis=0, keepdims=True)
    y43 = y4.reshape(tn, kk, do)
    mxs = [y43[:, j, :] for j in range(kk)]
    mns = mxs
    while len(mxs) > 1:
        mxs = [jnp.maximum(mxs[2 * t], mxs[2 * t + 1])
               for t in range(len(mxs) // 2)]
        mns = [jnp.minimum(mns[2 * t], mns[2 * t + 1])
               for t in range(len(mns) // 2)]
    mx_ref[...] = mxs[0]
    mn_ref[...] = mns[0]

    @pl.when(i == 0)
    def _():
        s4_ref[...] = s
        ss4_ref[...] = ss

    @pl.when(i > 0)
    def _():
        s4_ref[...] += s
        ss4_ref[...] += ss


def _fin_body(mx_ref, mn_ref, sc_ref, sh_ref, o_ref):
    sc = sc_ref[...]
    val = jnp.where(sc >= 0.0, mx_ref[...], mn_ref[...])
    o_ref[...] = jnp.maximum(val * sc + sh_ref[...], 0.0)


def _sc_gather(idx4, h2d, dh):
    nw, sup, inner, ch = 32, 64, 4, 128
    mesh = plsc.VectorSubcoreMesh(core_axis_name="c", subcore_axis_name="s")

    @functools.partial(
        pl.kernel, mesh=mesh,
        out_type=jax.ShapeDtypeStruct((nw, sup, inner, ch, dh), jnp.float32),
        scratch_types=[
            pltpu.VMEM((sup * inner, ch), jnp.int32),
            pltpu.VMEM((inner, ch, dh), jnp.float32),
            pltpu.SemaphoreType.DMA,
        ],
    )
    def gather_k(idx_hbm, tab_hbm, out_hbm, idx_v, buf_v, sem):
        wid = lax.axis_index("s") * 2 + lax.axis_index("c")
        pltpu.sync_copy(idx_hbm.at[wid], idx_v)

        def body(scn, carry):
            descs = [
                pltpu.async_copy(tab_hbm.at[idx_v.at[scn * inner + j]],
                                 buf_v.at[j], sem)
                for j in range(inner)
            ]
            for dsc in descs:
                dsc.wait()
            pltpu.sync_copy(buf_v, out_hbm.at[wid, scn])
            return carry

        lax.fori_loop(0, sup, body, 0)

    return gather_k(idx4, h2d)


def kernel(x, w1, b1, g1, be1, w2, b2, g2, be2, lw1, lg1, lb1, lw2, lg2, lb2):
    bsz, n, din = x.shape
    dh = w1.shape[0]
    do = lw1.shape[0]
    kk = 32
    m = bsz * n
    nm = m * kk
    r = 1024
    tn = 256
    tk = tn * kk

    x2d = x.reshape(m, din)

    y1a, s1a = pl.pallas_call(
        _y1_body,
        grid=(m // r,),
        in_specs=[pl.BlockSpec((r, din), lambda i: (i, 0)),
                  pl.BlockSpec((dh, din), lambda i: (0, 0)),
                  pl.BlockSpec((1, dh), lambda i: (0, 0))],
        out_specs=[pl.BlockSpec((r, dh), lambda i: (i, 0)),
                   pl.BlockSpec((1, dh), lambda i: (0, 0))],
        out_shape=[jax.ShapeDtypeStruct((m, dh), jnp.float32),
                   jax.ShapeDtypeStruct((1, dh), jnp.float32)],
    )(x2d, w1, b1[None, :])
    m1 = s1a / m

    sv1 = pl.pallas_call(
        _var_body,
        grid=(m // r,),
        in_specs=[pl.BlockSpec((r, dh), lambda i: (i, 0)),
                  pl.BlockSpec((1, dh), lambda i: (0, 0))],
        out_specs=pl.BlockSpec((1, dh), lambda i: (0, 0)),
        out_shape=jax.ShapeDtypeStruct((1, dh), jnp.float32),
    )(y1a, m1)
    v1 = sv1 / m
    sc1 = g1[None, :] / jnp.sqrt(v1 + _EPS)
    sh1 = be1[None, :] - m1 * sc1

    y2a, s2a = pl.pallas_call(
        _y2_body,
        grid=(m // r,),
        in_specs=[pl.BlockSpec((r, dh), lambda i: (i, 0)),
                  pl.BlockSpec((1, dh), lambda i: (0, 0)),
                  pl.BlockSpec((1, dh), lambda i: (0, 0)),
                  pl.BlockSpec((dh, dh), lambda i: (0, 0)),
                  pl.BlockSpec((1, dh), lambda i: (0, 0))],
        out_specs=[pl.BlockSpec((r, dh), lambda i: (i, 0)),
                   pl.BlockSpec((1, dh), lambda i: (0, 0))],
        out_shape=[jax.ShapeDtypeStruct((m, dh), jnp.float32),
                   jax.ShapeDtypeStruct((1, dh), jnp.float32)],
    )(y1a, sc1, sh1, w2, b2[None, :])
    m2 = s2a / m

    sv2 = pl.pallas_call(
        _var_body,
        grid=(m // r,),
        in_specs=[pl.BlockSpec((r, dh), lambda i: (i, 0)),
                  pl.BlockSpec((1, dh), lambda i: (0, 0))],
        out_specs=pl.BlockSpec((1, dh), lambda i: (0, 0)),
        out_shape=jax.ShapeDtypeStruct((1, dh), jnp.float32),
    )(y2a, m2)
    v2 = sv2 / m
    sc2 = g2[None, :] / jnp.sqrt(v2 + _EPS)
    sh2 = be2[None, :] - m2 * sc2

    a_mat = lw1[:, :dh]
    bm_mat = lw1[:, dh:] - lw1[:, :dh]

    p2d, q = pl.pallas_call(
        _pq_body,
        grid=(m // r,),
        in_specs=[pl.BlockSpec((r, dh), lambda i: (i, 0)),
                  pl.BlockSpec((1, dh), lambda i: (0, 0)),
                  pl.BlockSpec((1, dh), lambda i: (0, 0)),
                  pl.BlockSpec((do, dh), lambda i: (0, 0)),
                  pl.BlockSpec((do, dh), lambda i: (0, 0))],
        out_specs=[pl.BlockSpec((r, do), lambda i: (i, 0)),
                   pl.BlockSpec((r, do), lambda i: (i, 0))],
        out_shape=[jax.ShapeDtypeStruct((m, do), jnp.float32),
                   jax.ShapeDtypeStruct((m, do), jnp.float32)],
    )(y2a, sc2, sh2, a_mat, bm_mat)

    idxg = pl.pallas_call(
        functools.partial(_topk_body, tn=tn, n=n, kk=kk),
        grid=(bsz, n // tn),
        in_specs=[pl.BlockSpec((1, tn, din), lambda b, t: (b, t, 0)),
                  pl.BlockSpec((1, n, din), lambda b, t: (b, 0, 0))],
        out_specs=pl.BlockSpec((1, tn, kk), lambda b, t: (b, t, 0)),
        out_shape=jax.ShapeDtypeStruct((bsz, n, kk), jnp.int32),
    )(x, x)

    idx4 = idxg.reshape(32, 256, 128)
    gh = _sc_gather(idx4, p2d, do).reshape(nm, do)

    s3, ss3 = pl.pallas_call(
        functools.partial(_s3_body, tn=tn, kk=kk, do=do),
        grid=(nm // tk,),
        in_specs=[pl.BlockSpec((tk, do), lambda i: (i, 0)),
                  pl.BlockSpec((tn, do), lambda i: (i, 0))],
        out_specs=[pl.BlockSpec((1, do), lambda i: (0, 0)),
                   pl.BlockSpec((1, do), lambda i: (0, 0))],
        out_shape=[jax.ShapeDtypeStruct((1, do), jnp.float32),
                   jax.ShapeDtypeStruct((1, do), jnp.float32)],
    )(gh, q)

    mean3 = s3[0] / nm
    var3 = ss3[0] / nm - mean3 * mean3
    sc3 = (lg1 / jnp.sqrt(var3 + _EPS))[None, :]
    sh3 = (lb1 - mean3 * sc3[0])[None, :]

    mxa, mna, s4, ss4 = pl.pallas_call(
        functools.partial(_main_body, tn=tn, kk=kk, do=do),
        grid=(nm // tk,),
        in_specs=[pl.BlockSpec((tk, do), lambda i: (i, 0)),
                  pl.BlockSpec((tn, do), lambda i: (i, 0)),
                  pl.BlockSpec((1, do), lambda i: (0, 0)),
                  pl.BlockSpec((1, do), lambda i: (0, 0)),
                  pl.BlockSpec((do, do), lambda i: (0, 0))],
        out_specs=[pl.BlockSpec((tn, do), lambda i: (i, 0)),
                   pl.BlockSpec((tn, do), lambda i: (i, 0)),
                   pl.BlockSpec((1, do), lambda i: (0, 0)),
                   pl.BlockSpec((1, do), lambda i: (0, 0))],
        out_shape=[jax.ShapeDtypeStruct((m, do), jnp.float32),
                   jax.ShapeDtypeStruct((m, do), jnp.float32),
                   jax.ShapeDtypeStruct((1, do), jnp.float32),
                   jax.ShapeDtypeStruct((1, do), jnp.float32)],
    )(gh, q, sc3, sh3, lw2)

    mean4 = s4[0] / nm
    var4 = ss4[0] / nm - mean4 * mean4
    sc4 = (lg2 / jnp.sqrt(var4 + _EPS))[None, :]
    sh4 = (lb2 - mean4 * sc4[0])[None, :]

    out = pl.pallas_call(
        _fin_body,
        grid=(m // r,),
        in_specs=[pl.BlockSpec((r, do), lambda i: (i, 0)),
                  pl.BlockSpec((r, do), lambda i: (i, 0)),
                  pl.BlockSpec((1, do), lambda i: (0, 0)),
                  pl.BlockSpec((1, do), lambda i: (0, 0))],
        out_specs=pl.BlockSpec((r, do), lambda i: (i, 0)),
        out_shape=jax.ShapeDtypeStruct((m, do), jnp.float32),
    )(mxa, mna, sc4, sh4)

    return out.reshape(bsz, n, do)
